# Initial kernel scaffold; baseline (speedup 1.0000x reference)
import jax
import jax.numpy as jnp
from jax.experimental import pallas as pl


def kernel(x, edge_index, edge_attr, W, att_src, att_dst, W_edge, att_edge, bias, ln_gamma, ln_beta):
    raise NotImplementedError("write your pallas kernel here")


# speedup vs baseline: 52.3212x; 52.3212x over previous
import functools
import jax
import jax.numpy as jnp
from jax import lax
from jax.experimental import pallas as pl
from jax.experimental.pallas import tpu as pltpu
from jax.experimental.pallas import tpu_sc as plsc

N = 10000
E = 160000
D = 256
H = 8
C = 32
HALF = 128
B = 400
NB = N // B
EPT = E // 16
K = 80
NCHUNK = EPT // K
NPAD = 10240
ROWS_PT = NPAD // 16
ZR = 128
PB = 80
NBP = N // PB


def _pre_body(x_ref, w_ref, asrc_ref, adst_ref, wedge_ref, aedge_ref,
              seg_ref, xp2_ref, av2_ref, coef_ref):
    xp = jnp.dot(x_ref[...], w_ref[...], preferred_element_type=jnp.float32)
    xp2_ref[0] = xp[:, :HALF]
    xp2_ref[1] = xp[:, HALF:]
    seg = seg_ref[...]
    a_s = jnp.dot(xp * asrc_ref[...], seg,
                  preferred_element_type=jnp.float32)
    a_d = jnp.dot(xp * adst_ref[...], seg,
                  preferred_element_type=jnp.float32)
    z8 = jnp.zeros((B, 8), jnp.float32)
    av2_ref[0] = jnp.concatenate([a_s, z8], axis=1)
    av2_ref[1] = jnp.concatenate([a_d, z8], axis=1)
    cf = jnp.dot(wedge_ref[...] * aedge_ref[...], seg,
                 preferred_element_type=jnp.float32)
    coef_ref[...] = jnp.concatenate([cf, jnp.zeros((1, 8), jnp.float32)],
                                    axis=1)


def _pre(x, W, asrc_flat, adst_flat, wedge, aedge_flat, seg):
    return pl.pallas_call(
        _pre_body,
        grid=(NB,),
        in_specs=[
            pl.BlockSpec((B, D), lambda i: (i, 0)),
            pl.BlockSpec((D, D), lambda i: (0, 0)),
            pl.BlockSpec((1, D), lambda i: (0, 0)),
            pl.BlockSpec((1, D), lambda i: (0, 0)),
            pl.BlockSpec((1, D), lambda i: (0, 0)),
            pl.BlockSpec((1, D), lambda i: (0, 0)),
            pl.BlockSpec((D, H), lambda i: (0, 0)),
        ],
        out_specs=[
            pl.BlockSpec((2, B, HALF), lambda i: (0, i, 0)),
            pl.BlockSpec((2, B, 16), lambda i: (0, i, 0)),
            pl.BlockSpec((1, 16), lambda i: (0, 0)),
        ],
        out_shape=[
            jax.ShapeDtypeStruct((2, N, HALF), jnp.float32),
            jax.ShapeDtypeStruct((2, N, 16), jnp.float32),
            jax.ShapeDtypeStruct((1, 16), jnp.float32),
        ],
    )(x, W, asrc_flat, adst_flat, wedge, aedge_flat, seg)


def _easum_body(ea_ref, out_ref):
    out_ref[...] = (jnp.sum(ea_ref[...]) * (1.0 / E)).reshape(1, 1)


def _eamean(ea2):
    return pl.pallas_call(
        _easum_body,
        out_shape=jax.ShapeDtypeStruct((1, 1), jnp.float32),
    )(ea2)


def _edge_body(src_hbm, dst_hbm, ea_hbm, at2_hbm, xp2_hbm, coef_hbm,
               accb, denb,
               sidx, didx, sidx_f, didx_n, eav, eac, asr, adr, wbuf, xq,
               coefv, zbuf, zbufd, acc, den, sem):
    c = lax.axis_index("c")
    s = lax.axis_index("s")
    row0 = s * ROWS_PT

    def zb(i, cry):
        for j in range(HALF // 16):
            zbuf[i, pl.ds(j * 16, 16)] = jnp.zeros((16,), jnp.float32)
        zbufd[i, :] = jnp.zeros((16,), jnp.float32)
        return cry
    lax.fori_loop(0, ZR, zb, 0)
    for t in range(ROWS_PT // ZR):
        pltpu.sync_copy(zbuf, acc.at[pl.ds(row0 + t * ZR, ZR)])
        pltpu.sync_copy(zbufd, den.at[pl.ds(row0 + t * ZR, ZR)])
    pltpu.sync_copy(coef_hbm, coefv)
    plsc.subcore_barrier()

    ebase = s * EPT
    cN = c * N
    cNP = c * NPAD
    hb = 4 * c

    def chunk(t, cry):
        off = ebase + t * K
        pltpu.sync_copy(src_hbm.at[pl.ds(off, K)], sidx)
        pltpu.sync_copy(dst_hbm.at[pl.ds(off, K)], didx)
        pltpu.sync_copy(ea_hbm.at[pl.ds(off, K)], eav)
        for j in range(K // 16):
            sl = pl.ds(j * 16, 16)
            sidx_f[sl] = sidx[sl] + cN
            didx_n[sl] = didx[sl] + N
        pltpu.async_copy(at2_hbm.at[sidx], asr, sem).wait()
        pltpu.async_copy(at2_hbm.at[didx_n], adr, sem).wait()
        pltpu.async_copy(xp2_hbm.at[sidx_f], xq, sem).wait()
        coefr = coefv[:]
        for g in range(K // 16):
            ea16 = eav[pl.ds(g * 16, 16)]
            for k2 in range(16):
                eac[g * 16 + k2, :] = ea16[k2] * coefr

        ivecs = [jnp.full((16,), hb + t, jnp.int32) for t in range(4)]

        def ebody(k, cry2):
            z = asr[k, :] + adr[k, :] + eac[k, :]
            z = jnp.maximum(z, 0.2 * z)
            w = jnp.exp(z)
            wbuf[k, :] = w
            for t in range(4):
                sv = w.at[ivecs[t]].get(mode="promise_in_bounds")
                for j in (2 * t, 2 * t + 1):
                    xsl = pl.ds(j * 16, 16)
                    xq[k, xsl] = xq[k, xsl] * sv
            return cry2
        lax.fori_loop(0, K, ebody, 0)
        pltpu.sync_copy(wbuf, den.at[didx], add=True)
        pltpu.sync_copy(xq, acc.at[didx], add=True)
        return cry
    lax.fori_loop(0, NCHUNK, chunk, 0)
    plsc.subcore_barrier()

    pltpu.sync_copy(acc.at[pl.ds(row0, ROWS_PT)],
                    accb.at[pl.ds(cNP + row0, ROWS_PT)])
    pltpu.sync_copy(den.at[pl.ds(row0, ROWS_PT)],
                    denb.at[pl.ds(cNP + row0, ROWS_PT)])


def _edge(src, dst, ea, at2, xp2, coef16):
    mesh = plsc.VectorSubcoreMesh(core_axis_name="c", subcore_axis_name="s")
    f = pl.kernel(
        _edge_body, mesh=mesh,
        compiler_params=pltpu.CompilerParams(use_tc_tiling_on_sc=False),
        out_type=[
            jax.ShapeDtypeStruct((2 * NPAD, HALF), jnp.float32),
            jax.ShapeDtypeStruct((2 * NPAD, 16), jnp.float32),
        ],
        scratch_types=[
            pltpu.VMEM((K,), jnp.int32),
            pltpu.VMEM((K,), jnp.int32),
            pltpu.VMEM((K,), jnp.int32),
            pltpu.VMEM((K,), jnp.int32),
            pltpu.VMEM((K,), jnp.float32),
            pltpu.VMEM((K, 16), jnp.float32),
            pltpu.VMEM((K, 16), jnp.float32),
            pltpu.VMEM((K, 16), jnp.float32),
            pltpu.VMEM((K, 16), jnp.float32),
            pltpu.VMEM((K, HALF), jnp.float32),
            pltpu.VMEM((16,), jnp.float32),
            pltpu.VMEM((ZR, HALF), jnp.float32),
            pltpu.VMEM((ZR, 16), jnp.float32),
            pltpu.VMEM_SHARED((NPAD, HALF), jnp.float32),
            pltpu.VMEM_SHARED((NPAD, 16), jnp.float32),
            pltpu.SemaphoreType.DMA,
        ],
    )
    return f(src, dst, ea, at2, xp2, coef16)


def _post_body(x_ref, xplo_ref, xphi_ref, avs_ref, avd_ref,
               acclo_ref, acchi_ref, den_ref, coef_ref, eam_ref,
               segt_ref, bias_ref, g_ref, b_ref, out_ref):
    a_s = avs_ref[...][:, :H]
    a_d = avd_ref[...][:, :H]
    al = a_s + a_d + eam_ref[0, 0] * coef_ref[...][:, :H]
    wl = jnp.exp(jnp.maximum(al, 0.2 * al))
    segt = segt_ref[...]
    wl32 = jnp.dot(wl, segt, preferred_element_type=jnp.float32)
    den8 = den_ref[...][:, :H] + wl
    den32 = jnp.dot(den8, segt, preferred_element_type=jnp.float32)
    xp = jnp.concatenate([xplo_ref[...], xphi_ref[...]], axis=1)
    acc = jnp.concatenate([acclo_ref[...], acchi_ref[...]], axis=1)
    acc = acc + wl32 * xp
    h = acc / den32 + bias_ref[...] + x_ref[...]
    mu = jnp.mean(h, axis=1, keepdims=True)
    var = jnp.mean((h - mu) ** 2, axis=1, keepdims=True)
    hn = (h - mu) * lax.rsqrt(var + 1e-5) * g_ref[...] + b_ref[...]
    out_ref[...] = jnp.maximum(hn, 0.0)


def _post(x, xp2, av2, accb, denb, coef16, eam, segt, bias, g, b):
    hoff_xp = N // PB
    hoff_acc = NPAD // PB
    return pl.pallas_call(
        _post_body,
        grid=(NBP,),
        in_specs=[
            pl.BlockSpec((PB, D), lambda i: (i, 0)),
            pl.BlockSpec((PB, HALF), lambda i: (i, 0)),
            pl.BlockSpec((PB, HALF), lambda i: (i + hoff_xp, 0)),
            pl.BlockSpec((PB, 16), lambda i: (i, 0)),
            pl.BlockSpec((PB, 16), lambda i: (i + hoff_xp, 0)),
            pl.BlockSpec((PB, HALF), lambda i: (i, 0)),
            pl.BlockSpec((PB, HALF), lambda i: (i + hoff_acc, 0)),
            pl.BlockSpec((PB, 16), lambda i: (i, 0)),
            pl.BlockSpec((1, 16), lambda i: (0, 0)),
            pl.BlockSpec((1, 1), lambda i: (0, 0)),
            pl.BlockSpec((H, D), lambda i: (0, 0)),
            pl.BlockSpec((1, D), lambda i: (0, 0)),
            pl.BlockSpec((1, D), lambda i: (0, 0)),
            pl.BlockSpec((1, D), lambda i: (0, 0)),
        ],
        out_specs=pl.BlockSpec((PB, D), lambda i: (i, 0)),
        out_shape=jax.ShapeDtypeStruct((N, D), jnp.float32),
    )(x, xp2, xp2, av2, av2, accb, accb, denb, coef16, eam, segt, bias, g, b)


def kernel(x, edge_index, edge_attr, W, att_src, att_dst, W_edge, att_edge,
           bias, ln_gamma, ln_beta):
    seg = (jnp.arange(D, dtype=jnp.int32)[:, None] // C ==
           jnp.arange(H, dtype=jnp.int32)[None, :]).astype(jnp.float32)
    asrc_flat = att_src.reshape(1, D)
    adst_flat = att_dst.reshape(1, D)
    aedge_flat = att_edge.reshape(1, D)

    xp2, av2, coef16 = _pre(x, W, asrc_flat, adst_flat, W_edge, aedge_flat,
                            seg)
    eam = _eamean(edge_attr.reshape(E // HALF, HALF))
    xp2f = xp2.reshape(2 * N, HALF)
    av2f = av2.reshape(2 * N, 16)
    accb, denb = _edge(edge_index[0], edge_index[1], edge_attr,
                       av2f, xp2f, coef16.reshape(16))
    return _post(x, xp2f, av2f, accb, denb, coef16, eam, seg.T,
                 bias.reshape(1, D), ln_gamma.reshape(1, D),
                 ln_beta.reshape(1, D))


# speedup vs baseline: 53.5685x; 1.0238x over previous
import functools
import jax
import jax.numpy as jnp
from jax import lax
from jax.experimental import pallas as pl
from jax.experimental.pallas import tpu as pltpu
from jax.experimental.pallas import tpu_sc as plsc

N = 10000
E = 160000
D = 256
H = 8
C = 32
HALF = 128
B = 400
NB = N // B
EPT = E // 16
K = 80
NCHUNK = EPT // K
NPAD = 10240
ROWS_PT = NPAD // 16
ZR = 128
PB = 80
NBP = N // PB


def _pre_body(x_ref, w_ref, asrc_ref, adst_ref, wedge_ref, aedge_ref,
              seg_ref, xp2_ref, av2_ref, coef_ref):
    xp = jnp.dot(x_ref[...], w_ref[...], preferred_element_type=jnp.float32)
    xp2_ref[0] = xp[:, :HALF]
    xp2_ref[1] = xp[:, HALF:]
    seg = seg_ref[...]
    a_s = jnp.dot(xp * asrc_ref[...], seg,
                  preferred_element_type=jnp.float32)
    a_d = jnp.dot(xp * adst_ref[...], seg,
                  preferred_element_type=jnp.float32)
    z8 = jnp.zeros((B, 8), jnp.float32)
    av2_ref[0] = jnp.concatenate([a_s, z8], axis=1)
    av2_ref[1] = jnp.concatenate([a_d, z8], axis=1)
    cf = jnp.dot(wedge_ref[...] * aedge_ref[...], seg,
                 preferred_element_type=jnp.float32)
    coef_ref[...] = jnp.concatenate([cf, jnp.zeros((1, 8), jnp.float32)],
                                    axis=1)


def _pre(x, W, asrc_flat, adst_flat, wedge, aedge_flat, seg):
    return pl.pallas_call(
        _pre_body,
        grid=(NB,),
        in_specs=[
            pl.BlockSpec((B, D), lambda i: (i, 0)),
            pl.BlockSpec((D, D), lambda i: (0, 0)),
            pl.BlockSpec((1, D), lambda i: (0, 0)),
            pl.BlockSpec((1, D), lambda i: (0, 0)),
            pl.BlockSpec((1, D), lambda i: (0, 0)),
            pl.BlockSpec((1, D), lambda i: (0, 0)),
            pl.BlockSpec((D, H), lambda i: (0, 0)),
        ],
        out_specs=[
            pl.BlockSpec((2, B, HALF), lambda i: (0, i, 0)),
            pl.BlockSpec((2, B, 16), lambda i: (0, i, 0)),
            pl.BlockSpec((1, 16), lambda i: (0, 0)),
        ],
        out_shape=[
            jax.ShapeDtypeStruct((2, N, HALF), jnp.float32),
            jax.ShapeDtypeStruct((2, N, 16), jnp.float32),
            jax.ShapeDtypeStruct((1, 16), jnp.float32),
        ],
    )(x, W, asrc_flat, adst_flat, wedge, aedge_flat, seg)


def _easum_body(ea_ref, out_ref):
    out_ref[...] = (jnp.sum(ea_ref[...]) * (1.0 / E)).reshape(1, 1)


def _eamean(ea2):
    return pl.pallas_call(
        _easum_body,
        out_shape=jax.ShapeDtypeStruct((1, 1), jnp.float32),
    )(ea2)


def _edge_body(src_hbm, dst_hbm, ea_hbm, at2_hbm, xp2_hbm, coef_hbm,
               accb, denb,
               sidx, didx, sidx_f, didx_n, eav, eac, asr, adr, wbuf, xq,
               coefv, zbuf, zbufd, acc, den, sem, sem2):
    c = lax.axis_index("c")
    s = lax.axis_index("s")
    row0 = s * ROWS_PT

    def zb(i, cry):
        for j in range(HALF // 16):
            zbuf[i, pl.ds(j * 16, 16)] = jnp.zeros((16,), jnp.float32)
        zbufd[i, :] = jnp.zeros((16,), jnp.float32)
        return cry
    lax.fori_loop(0, ZR, zb, 0)
    for t in range(ROWS_PT // ZR):
        pltpu.sync_copy(zbuf, acc.at[pl.ds(row0 + t * ZR, ZR)])
        pltpu.sync_copy(zbufd, den.at[pl.ds(row0 + t * ZR, ZR)])
    pltpu.sync_copy(coef_hbm, coefv)
    plsc.subcore_barrier()

    ebase = s * EPT
    cN = c * N
    cNP = c * NPAD
    hb = 4 * c

    def chunk(t, cry):
        off = ebase + t * K
        l1 = pltpu.async_copy(src_hbm.at[pl.ds(off, K)], sidx, sem)
        l2 = pltpu.async_copy(dst_hbm.at[pl.ds(off, K)], didx, sem)
        l3 = pltpu.async_copy(ea_hbm.at[pl.ds(off, K)], eav, sem)
        l1.wait()
        l2.wait()
        for j in range(K // 16):
            sl = pl.ds(j * 16, 16)
            sidx_f[sl] = sidx[sl] + cN
            didx_n[sl] = didx[sl] + N
        g1 = pltpu.async_copy(at2_hbm.at[sidx], asr, sem2)
        g2 = pltpu.async_copy(at2_hbm.at[didx_n], adr, sem2)
        g3 = pltpu.async_copy(xp2_hbm.at[sidx_f], xq, sem2)
        l3.wait()
        coefr = coefv[:]
        for g in range(K // 16):
            ea16 = eav[pl.ds(g * 16, 16)]
            for k2 in range(16):
                eac[g * 16 + k2, :] = ea16[k2] * coefr
        g1.wait()
        g2.wait()
        g3.wait()

        ivecs = [jnp.full((16,), hb + t, jnp.int32) for t in range(4)]

        def ebody(k, cry2):
            z = asr[k, :] + adr[k, :] + eac[k, :]
            z = jnp.maximum(z, 0.2 * z)
            w = jnp.exp(z)
            wbuf[k, :] = w
            for t in range(4):
                sv = w.at[ivecs[t]].get(mode="promise_in_bounds")
                for j in (2 * t, 2 * t + 1):
                    xsl = pl.ds(j * 16, 16)
                    xq[k, xsl] = xq[k, xsl] * sv
            return cry2
        lax.fori_loop(0, K, ebody, 0, unroll=4)
        pltpu.sync_copy(wbuf, den.at[didx], add=True)
        pltpu.sync_copy(xq, acc.at[didx], add=True)
        return cry
    lax.fori_loop(0, NCHUNK, chunk, 0)
    plsc.subcore_barrier()

    pltpu.sync_copy(acc.at[pl.ds(row0, ROWS_PT)],
                    accb.at[pl.ds(cNP + row0, ROWS_PT)])
    pltpu.sync_copy(den.at[pl.ds(row0, ROWS_PT)],
                    denb.at[pl.ds(cNP + row0, ROWS_PT)])


def _edge(src, dst, ea, at2, xp2, coef16):
    mesh = plsc.VectorSubcoreMesh(core_axis_name="c", subcore_axis_name="s")
    f = pl.kernel(
        _edge_body, mesh=mesh,
        compiler_params=pltpu.CompilerParams(use_tc_tiling_on_sc=False),
        out_type=[
            jax.ShapeDtypeStruct((2 * NPAD, HALF), jnp.float32),
            jax.ShapeDtypeStruct((2 * NPAD, 16), jnp.float32),
        ],
        scratch_types=[
            pltpu.VMEM((K,), jnp.int32),
            pltpu.VMEM((K,), jnp.int32),
            pltpu.VMEM((K,), jnp.int32),
            pltpu.VMEM((K,), jnp.int32),
            pltpu.VMEM((K,), jnp.float32),
            pltpu.VMEM((K, 16), jnp.float32),
            pltpu.VMEM((K, 16), jnp.float32),
            pltpu.VMEM((K, 16), jnp.float32),
            pltpu.VMEM((K, 16), jnp.float32),
            pltpu.VMEM((K, HALF), jnp.float32),
            pltpu.VMEM((16,), jnp.float32),
            pltpu.VMEM((ZR, HALF), jnp.float32),
            pltpu.VMEM((ZR, 16), jnp.float32),
            pltpu.VMEM_SHARED((NPAD, HALF), jnp.float32),
            pltpu.VMEM_SHARED((NPAD, 16), jnp.float32),
            pltpu.SemaphoreType.DMA,
            pltpu.SemaphoreType.DMA,
        ],
    )
    return f(src, dst, ea, at2, xp2, coef16)


def _post_body(x_ref, xplo_ref, xphi_ref, avs_ref, avd_ref,
               acclo_ref, acchi_ref, den_ref, coef_ref, eam_ref,
               segt_ref, bias_ref, g_ref, b_ref, out_ref):
    a_s = avs_ref[...][:, :H]
    a_d = avd_ref[...][:, :H]
    al = a_s + a_d + eam_ref[0, 0] * coef_ref[...][:, :H]
    wl = jnp.exp(jnp.maximum(al, 0.2 * al))
    segt = segt_ref[...]
    wl32 = jnp.dot(wl, segt, preferred_element_type=jnp.float32)
    den8 = den_ref[...][:, :H] + wl
    den32 = jnp.dot(den8, segt, preferred_element_type=jnp.float32)
    xp = jnp.concatenate([xplo_ref[...], xphi_ref[...]], axis=1)
    acc = jnp.concatenate([acclo_ref[...], acchi_ref[...]], axis=1)
    acc = acc + wl32 * xp
    h = acc / den32 + bias_ref[...] + x_ref[...]
    mu = jnp.mean(h, axis=1, keepdims=True)
    var = jnp.mean((h - mu) ** 2, axis=1, keepdims=True)
    hn = (h - mu) * lax.rsqrt(var + 1e-5) * g_ref[...] + b_ref[...]
    out_ref[...] = jnp.maximum(hn, 0.0)


def _post(x, xp2, av2, accb, denb, coef16, eam, segt, bias, g, b):
    hoff_xp = N // PB
    hoff_acc = NPAD // PB
    return pl.pallas_call(
        _post_body,
        grid=(NBP,),
        in_specs=[
            pl.BlockSpec((PB, D), lambda i: (i, 0)),
            pl.BlockSpec((PB, HALF), lambda i: (i, 0)),
            pl.BlockSpec((PB, HALF), lambda i: (i + hoff_xp, 0)),
            pl.BlockSpec((PB, 16), lambda i: (i, 0)),
            pl.BlockSpec((PB, 16), lambda i: (i + hoff_xp, 0)),
            pl.BlockSpec((PB, HALF), lambda i: (i, 0)),
            pl.BlockSpec((PB, HALF), lambda i: (i + hoff_acc, 0)),
            pl.BlockSpec((PB, 16), lambda i: (i, 0)),
            pl.BlockSpec((1, 16), lambda i: (0, 0)),
            pl.BlockSpec((1, 1), lambda i: (0, 0)),
            pl.BlockSpec((H, D), lambda i: (0, 0)),
            pl.BlockSpec((1, D), lambda i: (0, 0)),
            pl.BlockSpec((1, D), lambda i: (0, 0)),
            pl.BlockSpec((1, D), lambda i: (0, 0)),
        ],
        out_specs=pl.BlockSpec((PB, D), lambda i: (i, 0)),
        out_shape=jax.ShapeDtypeStruct((N, D), jnp.float32),
    )(x, xp2, xp2, av2, av2, accb, accb, denb, coef16, eam, segt, bias, g, b)


def kernel(x, edge_index, edge_attr, W, att_src, att_dst, W_edge, att_edge,
           bias, ln_gamma, ln_beta):
    seg = (jnp.arange(D, dtype=jnp.int32)[:, None] // C ==
           jnp.arange(H, dtype=jnp.int32)[None, :]).astype(jnp.float32)
    asrc_flat = att_src.reshape(1, D)
    adst_flat = att_dst.reshape(1, D)
    aedge_flat = att_edge.reshape(1, D)

    xp2, av2, coef16 = _pre(x, W, asrc_flat, adst_flat, W_edge, aedge_flat,
                            seg)
    eam = _eamean(edge_attr.reshape(E // HALF, HALF))
    xp2f = xp2.reshape(2 * N, HALF)
    av2f = av2.reshape(2 * N, 16)
    accb, denb = _edge(edge_index[0], edge_index[1], edge_attr,
                       av2f, xp2f, coef16.reshape(16))
    return _post(x, xp2f, av2f, accb, denb, coef16, eam, seg.T,
                 bias.reshape(1, D), ln_gamma.reshape(1, D),
                 ln_beta.reshape(1, D))


# speedup vs baseline: 94.9340x; 1.7722x over previous
import functools
import jax
import jax.numpy as jnp
from jax import lax
from jax.experimental import pallas as pl
from jax.experimental.pallas import tpu as pltpu
from jax.experimental.pallas import tpu_sc as plsc

N = 10000
E = 160000
D = 256
H = 8
C = 32
HALF = 128
B = 400
NB = N // B
EPT = E // 16
K = 80
NCHUNK = EPT // K
NPAD = 10240
ROWS_PT = NPAD // 16
ZR = 128
PB = 80
NBP = N // PB


def _pre_body(x_ref, w_ref, asrc_ref, adst_ref, wedge_ref, aedge_ref,
              seg_ref, xp2_ref, av2_ref, coef_ref):
    xp = jnp.dot(x_ref[...], w_ref[...], preferred_element_type=jnp.float32)
    xp2_ref[0] = xp[:, :HALF]
    xp2_ref[1] = xp[:, HALF:]
    seg = seg_ref[...]
    a_s = jnp.dot(xp * asrc_ref[...], seg,
                  preferred_element_type=jnp.float32)
    a_d = jnp.dot(xp * adst_ref[...], seg,
                  preferred_element_type=jnp.float32)
    z8 = jnp.zeros((B, 8), jnp.float32)
    av2_ref[0] = jnp.concatenate([a_s, z8], axis=1)
    av2_ref[1] = jnp.concatenate([a_d, z8], axis=1)
    cf = jnp.dot(wedge_ref[...] * aedge_ref[...], seg,
                 preferred_element_type=jnp.float32)
    coef_ref[...] = jnp.concatenate([cf, jnp.zeros((1, 8), jnp.float32)],
                                    axis=1)


def _pre(x, W, asrc_flat, adst_flat, wedge, aedge_flat, seg):
    return pl.pallas_call(
        _pre_body,
        grid=(NB,),
        in_specs=[
            pl.BlockSpec((B, D), lambda i: (i, 0)),
            pl.BlockSpec((D, D), lambda i: (0, 0)),
            pl.BlockSpec((1, D), lambda i: (0, 0)),
            pl.BlockSpec((1, D), lambda i: (0, 0)),
            pl.BlockSpec((1, D), lambda i: (0, 0)),
            pl.BlockSpec((1, D), lambda i: (0, 0)),
            pl.BlockSpec((D, H), lambda i: (0, 0)),
        ],
        out_specs=[
            pl.BlockSpec((2, B, HALF), lambda i: (0, i, 0)),
            pl.BlockSpec((2, B, 16), lambda i: (0, i, 0)),
            pl.BlockSpec((1, 16), lambda i: (0, 0)),
        ],
        out_shape=[
            jax.ShapeDtypeStruct((2, N, HALF), jnp.float32),
            jax.ShapeDtypeStruct((2, N, 16), jnp.float32),
            jax.ShapeDtypeStruct((1, 16), jnp.float32),
        ],
    )(x, W, asrc_flat, adst_flat, wedge, aedge_flat, seg)


def _easum_body(ea_ref, out_ref):
    out_ref[...] = (jnp.sum(ea_ref[...]) * (1.0 / E)).reshape(1, 1)


def _eamean(ea2):
    return pl.pallas_call(
        _easum_body,
        out_shape=jax.ShapeDtypeStruct((1, 1), jnp.float32),
    )(ea2)


def _edge_body(src_hbm, dst_hbm, ea_hbm, at2_hbm, xp2_hbm, coef_hbm,
               accb, denb,
               sidx, didx, sidx_f, didx_n, eav, eac, asr, adr, wbuf, xq,
               coefv, zbuf, zbufd, acc, den, sem, sem2):
    c = lax.axis_index("c")
    s = lax.axis_index("s")
    row0 = s * ROWS_PT

    def zb(i, cry):
        for j in range(HALF // 16):
            zbuf[i, pl.ds(j * 16, 16)] = jnp.zeros((16,), jnp.float32)
        zbufd[i, :] = jnp.zeros((16,), jnp.float32)
        return cry
    lax.fori_loop(0, ZR, zb, 0)
    for t in range(ROWS_PT // ZR):
        pltpu.sync_copy(zbuf, acc.at[pl.ds(row0 + t * ZR, ZR)])
        pltpu.sync_copy(zbufd, den.at[pl.ds(row0 + t * ZR, ZR)])
    pltpu.sync_copy(coef_hbm, coefv)
    plsc.subcore_barrier()

    ebase = s * EPT
    cN = c * N
    cNP = c * NPAD
    hb = 4 * c

    def chunk(t, cry):
        off = ebase + t * K
        l1 = pltpu.async_copy(src_hbm.at[pl.ds(off, K)], sidx, sem)
        l2 = pltpu.async_copy(dst_hbm.at[pl.ds(off, K)], didx, sem)
        l3 = pltpu.async_copy(ea_hbm.at[pl.ds(off, K)], eav, sem)
        l1.wait()
        l2.wait()
        for j in range(K // 16):
            sl = pl.ds(j * 16, 16)
            sidx_f[sl] = sidx[sl] + cN
            didx_n[sl] = didx[sl] + N
        g1 = pltpu.async_copy(at2_hbm.at[sidx], asr, sem2)
        g2 = pltpu.async_copy(at2_hbm.at[didx_n], adr, sem2)
        g3 = pltpu.async_copy(xp2_hbm.at[sidx_f], xq, sem2)
        l3.wait()
        coefr = coefv[:]
        for g in range(K // 16):
            ea16 = eav[pl.ds(g * 16, 16)]
            for k2 in range(16):
                eac[g * 16 + k2, :] = ea16[k2] * coefr
        g1.wait()
        g2.wait()
        g3.wait()

        ivecs = [jnp.full((16,), hb + t, jnp.int32) for t in range(4)]

        @plsc.parallel_loop(0, K, step=1, unroll=4)
        def ebody(k):
            z = asr[k, :] + adr[k, :] + eac[k, :]
            z = jnp.maximum(z, 0.2 * z)
            w = jnp.exp(z)
            wbuf[k, :] = w
            for t in range(4):
                sv = w.at[ivecs[t]].get(mode="promise_in_bounds")
                for j in (2 * t, 2 * t + 1):
                    xsl = pl.ds(j * 16, 16)
                    xq[k, xsl] = xq[k, xsl] * sv
        pltpu.sync_copy(wbuf, den.at[didx], add=True)
        pltpu.sync_copy(xq, acc.at[didx], add=True)
        return cry
    lax.fori_loop(0, NCHUNK, chunk, 0)
    plsc.subcore_barrier()

    pltpu.sync_copy(acc.at[pl.ds(row0, ROWS_PT)],
                    accb.at[pl.ds(cNP + row0, ROWS_PT)])
    pltpu.sync_copy(den.at[pl.ds(row0, ROWS_PT)],
                    denb.at[pl.ds(cNP + row0, ROWS_PT)])


def _edge(src, dst, ea, at2, xp2, coef16):
    mesh = plsc.VectorSubcoreMesh(core_axis_name="c", subcore_axis_name="s")
    f = pl.kernel(
        _edge_body, mesh=mesh,
        compiler_params=pltpu.CompilerParams(use_tc_tiling_on_sc=False),
        out_type=[
            jax.ShapeDtypeStruct((2 * NPAD, HALF), jnp.float32),
            jax.ShapeDtypeStruct((2 * NPAD, 16), jnp.float32),
        ],
        scratch_types=[
            pltpu.VMEM((K,), jnp.int32),
            pltpu.VMEM((K,), jnp.int32),
            pltpu.VMEM((K,), jnp.int32),
            pltpu.VMEM((K,), jnp.int32),
            pltpu.VMEM((K,), jnp.float32),
            pltpu.VMEM((K, 16), jnp.float32),
            pltpu.VMEM((K, 16), jnp.float32),
            pltpu.VMEM((K, 16), jnp.float32),
            pltpu.VMEM((K, 16), jnp.float32),
            pltpu.VMEM((K, HALF), jnp.float32),
            pltpu.VMEM((16,), jnp.float32),
            pltpu.VMEM((ZR, HALF), jnp.float32),
            pltpu.VMEM((ZR, 16), jnp.float32),
            pltpu.VMEM_SHARED((NPAD, HALF), jnp.float32),
            pltpu.VMEM_SHARED((NPAD, 16), jnp.float32),
            pltpu.SemaphoreType.DMA,
            pltpu.SemaphoreType.DMA,
        ],
    )
    return f(src, dst, ea, at2, xp2, coef16)


def _post_body(x_ref, xplo_ref, xphi_ref, avs_ref, avd_ref,
               acclo_ref, acchi_ref, den_ref, coef_ref, eam_ref,
               segt_ref, bias_ref, g_ref, b_ref, out_ref):
    a_s = avs_ref[...][:, :H]
    a_d = avd_ref[...][:, :H]
    al = a_s + a_d + eam_ref[0, 0] * coef_ref[...][:, :H]
    wl = jnp.exp(jnp.maximum(al, 0.2 * al))
    segt = segt_ref[...]
    wl32 = jnp.dot(wl, segt, preferred_element_type=jnp.float32)
    den8 = den_ref[...][:, :H] + wl
    den32 = jnp.dot(den8, segt, preferred_element_type=jnp.float32)
    xp = jnp.concatenate([xplo_ref[...], xphi_ref[...]], axis=1)
    acc = jnp.concatenate([acclo_ref[...], acchi_ref[...]], axis=1)
    acc = acc + wl32 * xp
    h = acc / den32 + bias_ref[...] + x_ref[...]
    mu = jnp.mean(h, axis=1, keepdims=True)
    var = jnp.mean((h - mu) ** 2, axis=1, keepdims=True)
    hn = (h - mu) * lax.rsqrt(var + 1e-5) * g_ref[...] + b_ref[...]
    out_ref[...] = jnp.maximum(hn, 0.0)


def _post(x, xp2, av2, accb, denb, coef16, eam, segt, bias, g, b):
    hoff_xp = N // PB
    hoff_acc = NPAD // PB
    return pl.pallas_call(
        _post_body,
        grid=(NBP,),
        in_specs=[
            pl.BlockSpec((PB, D), lambda i: (i, 0)),
            pl.BlockSpec((PB, HALF), lambda i: (i, 0)),
            pl.BlockSpec((PB, HALF), lambda i: (i + hoff_xp, 0)),
            pl.BlockSpec((PB, 16), lambda i: (i, 0)),
            pl.BlockSpec((PB, 16), lambda i: (i + hoff_xp, 0)),
            pl.BlockSpec((PB, HALF), lambda i: (i, 0)),
            pl.BlockSpec((PB, HALF), lambda i: (i + hoff_acc, 0)),
            pl.BlockSpec((PB, 16), lambda i: (i, 0)),
            pl.BlockSpec((1, 16), lambda i: (0, 0)),
            pl.BlockSpec((1, 1), lambda i: (0, 0)),
            pl.BlockSpec((H, D), lambda i: (0, 0)),
            pl.BlockSpec((1, D), lambda i: (0, 0)),
            pl.BlockSpec((1, D), lambda i: (0, 0)),
            pl.BlockSpec((1, D), lambda i: (0, 0)),
        ],
        out_specs=pl.BlockSpec((PB, D), lambda i: (i, 0)),
        out_shape=jax.ShapeDtypeStruct((N, D), jnp.float32),
    )(x, xp2, xp2, av2, av2, accb, accb, denb, coef16, eam, segt, bias, g, b)


def kernel(x, edge_index, edge_attr, W, att_src, att_dst, W_edge, att_edge,
           bias, ln_gamma, ln_beta):
    seg = (jnp.arange(D, dtype=jnp.int32)[:, None] // C ==
           jnp.arange(H, dtype=jnp.int32)[None, :]).astype(jnp.float32)
    asrc_flat = att_src.reshape(1, D)
    adst_flat = att_dst.reshape(1, D)
    aedge_flat = att_edge.reshape(1, D)

    xp2, av2, coef16 = _pre(x, W, asrc_flat, adst_flat, W_edge, aedge_flat,
                            seg)
    eam = _eamean(edge_attr.reshape(E // HALF, HALF))
    xp2f = xp2.reshape(2 * N, HALF)
    av2f = av2.reshape(2 * N, 16)
    accb, denb = _edge(edge_index[0], edge_index[1], edge_attr,
                       av2f, xp2f, coef16.reshape(16))
    return _post(x, xp2f, av2f, accb, denb, coef16, eam, seg.T,
                 bias.reshape(1, D), ln_gamma.reshape(1, D),
                 ln_beta.reshape(1, D))


# speedup vs baseline: 153.1649x; 1.6134x over previous
import functools
import jax
import jax.numpy as jnp
from jax import lax
from jax.experimental import pallas as pl
from jax.experimental.pallas import tpu as pltpu
from jax.experimental.pallas import tpu_sc as plsc

N = 10000
E = 160000
D = 256
H = 8
C = 32
HALF = 128
B = 400
NB = N // B
EPT = E // 16
K = 80
NCHUNK = EPT // K
NPAD = 10240
ROWS_PT = NPAD // 16
ZR = 128
PB = 80
NBP = N // PB


def _pre_body(x_ref, w_ref, asrc_ref, adst_ref, wedge_ref, aedge_ref,
              seg_ref, xp2_ref, av2_ref, coef_ref):
    xp = jnp.dot(x_ref[...], w_ref[...], preferred_element_type=jnp.float32)
    xp2_ref[0] = xp[:, :HALF]
    xp2_ref[1] = xp[:, HALF:]
    seg = seg_ref[...]
    a_s = jnp.dot(xp * asrc_ref[...], seg,
                  preferred_element_type=jnp.float32)
    a_d = jnp.dot(xp * adst_ref[...], seg,
                  preferred_element_type=jnp.float32)
    z8 = jnp.zeros((B, 8), jnp.float32)
    av2_ref[0] = jnp.concatenate([a_s, z8], axis=1)
    av2_ref[1] = jnp.concatenate([a_d, z8], axis=1)
    cf = jnp.dot(wedge_ref[...] * aedge_ref[...], seg,
                 preferred_element_type=jnp.float32)
    coef_ref[...] = jnp.concatenate([cf, jnp.zeros((1, 8), jnp.float32)],
                                    axis=1)


def _pre(x, W, asrc_flat, adst_flat, wedge, aedge_flat, seg):
    return pl.pallas_call(
        _pre_body,
        grid=(NB,),
        in_specs=[
            pl.BlockSpec((B, D), lambda i: (i, 0)),
            pl.BlockSpec((D, D), lambda i: (0, 0)),
            pl.BlockSpec((1, D), lambda i: (0, 0)),
            pl.BlockSpec((1, D), lambda i: (0, 0)),
            pl.BlockSpec((1, D), lambda i: (0, 0)),
            pl.BlockSpec((1, D), lambda i: (0, 0)),
            pl.BlockSpec((D, H), lambda i: (0, 0)),
        ],
        out_specs=[
            pl.BlockSpec((2, B, HALF), lambda i: (0, i, 0)),
            pl.BlockSpec((2, B, 16), lambda i: (0, i, 0)),
            pl.BlockSpec((1, 16), lambda i: (0, 0)),
        ],
        out_shape=[
            jax.ShapeDtypeStruct((2, N, HALF), jnp.float32),
            jax.ShapeDtypeStruct((2, N, 16), jnp.float32),
            jax.ShapeDtypeStruct((1, 16), jnp.float32),
        ],
    )(x, W, asrc_flat, adst_flat, wedge, aedge_flat, seg)


def _easum_body(ea_ref, out_ref):
    out_ref[...] = (jnp.sum(ea_ref[...]) * (1.0 / E)).reshape(1, 1)


def _eamean(ea2):
    return pl.pallas_call(
        _easum_body,
        out_shape=jax.ShapeDtypeStruct((1, 1), jnp.float32),
    )(ea2)


def _edge_body(src_hbm, dst_hbm, ea_hbm, at2_hbm, xp2_hbm, coef_hbm,
               accb, denb,
               sidx0, didx0, eav0, sfx0, dnx0, sd0, asr0, adr0, xq0, wbuf0,
               sidx1, didx1, eav1, sfx1, dnx1, sd1, asr1, adr1, xq1, wbuf1,
               eac, coefv, acc, den,
               semL0, semL1, semG0, semG1, semS0, semS1):
    c = lax.axis_index("c")
    s = lax.axis_index("s")
    row0 = s * ROWS_PT
    ebase = s * EPT
    cN = c * N
    cNP = c * NPAD
    hb = 4 * c

    set0 = (sidx0, didx0, eav0, sfx0, dnx0, sd0, asr0, adr0, xq0, wbuf0,
            semL0, semG0, semS0)
    set1 = (sidx1, didx1, eav1, sfx1, dnx1, sd1, asr1, adr1, xq1, wbuf1,
            semL1, semG1, semS1)

    pltpu.sync_copy(coef_hbm, coefv)
    pltpu.sync_copy(src_hbm.at[pl.ds(ebase, K)], sidx0)
    pltpu.sync_copy(dst_hbm.at[pl.ds(ebase, K)], didx0)
    pltpu.sync_copy(ea_hbm.at[pl.ds(ebase, K)], eav0)
    for j in range(K // 16):
        sl = pl.ds(j * 16, 16)
        sfx0[sl] = sidx0[sl] + cN
        dnx0[sl] = didx0[sl] + N

    def zb(i, cry):
        for j in range(HALF // 16):
            xq0[i, pl.ds(j * 16, 16)] = jnp.zeros((16,), jnp.float32)
        wbuf0[i, :] = jnp.zeros((16,), jnp.float32)
        return cry
    lax.fori_loop(0, K, zb, 0)
    for u in range(ROWS_PT // K):
        pltpu.sync_copy(xq0, acc.at[pl.ds(row0 + u * K, K)])
        pltpu.sync_copy(wbuf0, den.at[pl.ds(row0 + u * K, K)])
    plsc.subcore_barrier()

    coefr = coefv[:]
    ivecs = [jnp.full((16,), hb + t, jnp.int32) for t in range(4)]

    def issue_gathers(Sx, semX):
        (sidxX, didxX, eavX, sfxX, dnxX, sdX, asrX, adrX, xqX, wbufX,
         _, _, _) = Sx
        pltpu.async_copy(at2_hbm.at[sidxX], asrX, semX)
        pltpu.async_copy(at2_hbm.at[dnxX], adrX, semX)
        pltpu.async_copy(xp2_hbm.at[sfxX], xqX, semX)

    def wait_gathers(asrX, adrX, xqX, semX):
        pltpu.make_async_copy(at2_hbm.at[pl.ds(0, K)], asrX, semX).wait()
        pltpu.make_async_copy(at2_hbm.at[pl.ds(0, K)], adrX, semX).wait()
        pltpu.make_async_copy(xp2_hbm.at[pl.ds(0, K)], xqX, semX).wait()

    def wait_scatters(wbufX, xqX, semX):
        pltpu.make_async_copy(at2_hbm.at[pl.ds(0, K)], wbufX, semX).wait()
        pltpu.make_async_copy(xp2_hbm.at[pl.ds(0, K)], xqX, semX).wait()

    def pipe_step(t, C, Nx):
        (sidxC, didxC, eavC, sfxC, dnxC, sdC, asrC, adrC, xqC, wbufC,
         semLC, semGC, semSC) = C
        (sidxN, didxN, eavN, sfxN, dnxN, sdN, asrN, adrN, xqN, wbufN,
         semLN, semGN, semSN) = Nx

        @pl.when(t + 1 < NCHUNK)
        def _():
            pltpu.make_async_copy(src_hbm.at[pl.ds(0, K)], sidxN,
                                  semLN).wait()
            pltpu.make_async_copy(src_hbm.at[pl.ds(0, K)], didxN,
                                  semLN).wait()
            pltpu.make_async_copy(ea_hbm.at[pl.ds(0, K)], eavN,
                                  semLN).wait()
            for j in range(K // 16):
                sl = pl.ds(j * 16, 16)
                sfxN[sl] = sidxN[sl] + cN
                dnxN[sl] = didxN[sl] + N

            @pl.when(t > 0)
            def _():
                wait_scatters(wbufN, xqN, semSN)
            issue_gathers(Nx, semGN)

        for g in range(K // 16):
            sl = pl.ds(g * 16, 16)
            sdC[sl] = didxC[sl]
            ea16 = eavC[sl]
            for k2 in range(16):
                eac[g * 16 + k2, :] = ea16[k2] * coefr

        wait_gathers(asrC, adrC, xqC, semGC)

        @pl.when(t + 2 < NCHUNK)
        def _():
            o = ebase + (t + 2) * K
            pltpu.async_copy(src_hbm.at[pl.ds(o, K)], sidxC, semLC)
            pltpu.async_copy(dst_hbm.at[pl.ds(o, K)], didxC, semLC)
            pltpu.async_copy(ea_hbm.at[pl.ds(o, K)], eavC, semLC)

        @plsc.parallel_loop(0, K, step=1, unroll=4)
        def ebody(k):
            z = asrC[k, :] + adrC[k, :] + eac[k, :]
            z = jnp.maximum(z, 0.2 * z)
            w = jnp.exp(z)
            wbufC[k, :] = w
            for u in range(4):
                sv = w.at[ivecs[u]].get(mode="promise_in_bounds")
                for j in (2 * u, 2 * u + 1):
                    xsl = pl.ds(j * 16, 16)
                    xqC[k, xsl] = xqC[k, xsl] * sv

        pltpu.async_copy(wbufC, den.at[sdC], semSC, add=True)
        pltpu.async_copy(xqC, acc.at[sdC], semSC, add=True)

    issue_gathers(set0, semG0)
    o1 = ebase + K
    pltpu.async_copy(src_hbm.at[pl.ds(o1, K)], sidx1, semL1)
    pltpu.async_copy(dst_hbm.at[pl.ds(o1, K)], didx1, semL1)
    pltpu.async_copy(ea_hbm.at[pl.ds(o1, K)], eav1, semL1)

    def chunk(t, cry):
        @pl.when(t % 2 == 0)
        def _():
            pipe_step(t, set0, set1)

        @pl.when(t % 2 != 0)
        def _():
            pipe_step(t, set1, set0)
        return cry
    lax.fori_loop(0, NCHUNK, chunk, 0)
    wait_scatters(wbuf1, xq1, semS1)
    wait_scatters(wbuf0, xq0, semS0)
    plsc.subcore_barrier()

    pltpu.sync_copy(acc.at[pl.ds(row0, ROWS_PT)],
                    accb.at[pl.ds(cNP + row0, ROWS_PT)])
    pltpu.sync_copy(den.at[pl.ds(row0, ROWS_PT)],
                    denb.at[pl.ds(cNP + row0, ROWS_PT)])


def _edge(src, dst, ea, at2, xp2, coef16):
    mesh = plsc.VectorSubcoreMesh(core_axis_name="c", subcore_axis_name="s")
    f = pl.kernel(
        _edge_body, mesh=mesh,
        compiler_params=pltpu.CompilerParams(use_tc_tiling_on_sc=False),
        out_type=[
            jax.ShapeDtypeStruct((2 * NPAD, HALF), jnp.float32),
            jax.ShapeDtypeStruct((2 * NPAD, 16), jnp.float32),
        ],
        scratch_types=(
            [pltpu.VMEM((K,), jnp.int32),
             pltpu.VMEM((K,), jnp.int32),
             pltpu.VMEM((K,), jnp.float32),
             pltpu.VMEM((K,), jnp.int32),
             pltpu.VMEM((K,), jnp.int32),
             pltpu.VMEM((K,), jnp.int32),
             pltpu.VMEM((K, 16), jnp.float32),
             pltpu.VMEM((K, 16), jnp.float32),
             pltpu.VMEM((K, HALF), jnp.float32),
             pltpu.VMEM((K, 16), jnp.float32),
             ] * 2 +
            [pltpu.VMEM((K, 16), jnp.float32),
             pltpu.VMEM((16,), jnp.float32),
             pltpu.VMEM_SHARED((NPAD, HALF), jnp.float32),
             pltpu.VMEM_SHARED((NPAD, 16), jnp.float32),
             ] +
            [pltpu.SemaphoreType.DMA] * 6),
    )
    return f(src, dst, ea, at2, xp2, coef16)


def _post_body(x_ref, xplo_ref, xphi_ref, avs_ref, avd_ref,
               acclo_ref, acchi_ref, den_ref, coef_ref, eam_ref,
               segt_ref, bias_ref, g_ref, b_ref, out_ref):
    a_s = avs_ref[...][:, :H]
    a_d = avd_ref[...][:, :H]
    al = a_s + a_d + eam_ref[0, 0] * coef_ref[...][:, :H]
    wl = jnp.exp(jnp.maximum(al, 0.2 * al))
    segt = segt_ref[...]
    wl32 = jnp.dot(wl, segt, preferred_element_type=jnp.float32)
    den8 = den_ref[...][:, :H] + wl
    den32 = jnp.dot(den8, segt, preferred_element_type=jnp.float32)
    xp = jnp.concatenate([xplo_ref[...], xphi_ref[...]], axis=1)
    acc = jnp.concatenate([acclo_ref[...], acchi_ref[...]], axis=1)
    acc = acc + wl32 * xp
    h = acc / den32 + bias_ref[...] + x_ref[...]
    mu = jnp.mean(h, axis=1, keepdims=True)
    var = jnp.mean((h - mu) ** 2, axis=1, keepdims=True)
    hn = (h - mu) * lax.rsqrt(var + 1e-5) * g_ref[...] + b_ref[...]
    out_ref[...] = jnp.maximum(hn, 0.0)


def _post(x, xp2, av2, accb, denb, coef16, eam, segt, bias, g, b):
    hoff_xp = N // PB
    hoff_acc = NPAD // PB
    return pl.pallas_call(
        _post_body,
        grid=(NBP,),
        in_specs=[
            pl.BlockSpec((PB, D), lambda i: (i, 0)),
            pl.BlockSpec((PB, HALF), lambda i: (i, 0)),
            pl.BlockSpec((PB, HALF), lambda i: (i + hoff_xp, 0)),
            pl.BlockSpec((PB, 16), lambda i: (i, 0)),
            pl.BlockSpec((PB, 16), lambda i: (i + hoff_xp, 0)),
            pl.BlockSpec((PB, HALF), lambda i: (i, 0)),
            pl.BlockSpec((PB, HALF), lambda i: (i + hoff_acc, 0)),
            pl.BlockSpec((PB, 16), lambda i: (i, 0)),
            pl.BlockSpec((1, 16), lambda i: (0, 0)),
            pl.BlockSpec((1, 1), lambda i: (0, 0)),
            pl.BlockSpec((H, D), lambda i: (0, 0)),
            pl.BlockSpec((1, D), lambda i: (0, 0)),
            pl.BlockSpec((1, D), lambda i: (0, 0)),
            pl.BlockSpec((1, D), lambda i: (0, 0)),
        ],
        out_specs=pl.BlockSpec((PB, D), lambda i: (i, 0)),
        out_shape=jax.ShapeDtypeStruct((N, D), jnp.float32),
    )(x, xp2, xp2, av2, av2, accb, accb, denb, coef16, eam, segt, bias, g, b)


def kernel(x, edge_index, edge_attr, W, att_src, att_dst, W_edge, att_edge,
           bias, ln_gamma, ln_beta):
    seg = (jnp.arange(D, dtype=jnp.int32)[:, None] // C ==
           jnp.arange(H, dtype=jnp.int32)[None, :]).astype(jnp.float32)
    asrc_flat = att_src.reshape(1, D)
    adst_flat = att_dst.reshape(1, D)
    aedge_flat = att_edge.reshape(1, D)

    xp2, av2, coef16 = _pre(x, W, asrc_flat, adst_flat, W_edge, aedge_flat,
                            seg)
    eam = _eamean(edge_attr.reshape(E // HALF, HALF))
    xp2f = xp2.reshape(2 * N, HALF)
    av2f = av2.reshape(2 * N, 16)
    accb, denb = _edge(edge_index[0], edge_index[1], edge_attr,
                       av2f, xp2f, coef16.reshape(16))
    return _post(x, xp2f, av2f, accb, denb, coef16, eam, seg.T,
                 bias.reshape(1, D), ln_gamma.reshape(1, D),
                 ln_beta.reshape(1, D))
